# baseline (speedup 1.0000x reference)
import jax
import jax.numpy as jnp
from jax.experimental import pallas as pl
from jax.experimental.pallas import tpu as pltpu

HEIGHT = 128
WIDTH = 128
SIZE = 8
NEURONS = 16
NUM_W = WIDTH - (SIZE - 1)
NUM_H = HEIGHT - (SIZE - 1)
NUM_ZONES = NUM_H * NUM_W
PATCH = SIZE * SIZE
RB = NUM_W * NEURONS
NBUF = 12


def _tc_body(x_ref, w_hbm, o_ref, wbuf, sem):
    def cp(r, slot):
        return pltpu.make_async_copy(
            w_hbm.at[pl.ds(r * RB, RB)], wbuf.at[slot], sem.at[slot])

    for b in range(NBUF - 1):
        cp(b, b).start()

    def row_fn(r, carry):
        nxt = r + NBUF - 1

        @pl.when(nxt < NUM_H)
        def _():
            cp(nxt, jax.lax.rem(nxt, NBUF)).start()

        slot = jax.lax.rem(r, NBUF)
        cp(r, slot).wait()
        wv = wbuf[slot].reshape(NUM_W, NEURONS, PATCH)

        xs = x_ref[pl.ds(r, SIZE), :]
        segs = []
        for dr in range(SIZE):
            row = xs[dr:dr + 1, :]
            for dc in range(SIZE):
                segs.append(row[:, dc:dc + NUM_W])
        PT = jnp.concatenate(segs, axis=0)
        P = PT.T
        prod = wv * P[:, None, :]
        resp = jnp.sum(prod, axis=2)
        m = jnp.max(resp, axis=1, keepdims=True)
        o_ref[r] = jnp.where(resp >= m, resp, 0.0)
        return carry

    jax.lax.fori_loop(0, NUM_H, row_fn, 0)


def kernel(x, W):
    W2 = W.reshape(NUM_ZONES * NEURONS, PATCH)
    out = pl.pallas_call(
        _tc_body,
        in_specs=[
            pl.BlockSpec((HEIGHT, WIDTH), lambda: (0, 0)),
            pl.BlockSpec(memory_space=pl.ANY),
        ],
        out_specs=pl.BlockSpec((NUM_H, NUM_W, NEURONS), lambda: (0, 0, 0)),
        out_shape=jax.ShapeDtypeStruct((NUM_H, NUM_W, NEURONS), jnp.float32),
        scratch_shapes=[
            pltpu.VMEM((NBUF, RB, PATCH), jnp.float32),
            pltpu.SemaphoreType.DMA((NBUF,)),
        ],
    )(x, W2)
    return out.reshape(NUM_ZONES, NEURONS)


# speedup vs baseline: 1.0197x; 1.0197x over previous
import jax
import jax.numpy as jnp
from jax.experimental import pallas as pl
from jax.experimental.pallas import tpu as pltpu

HEIGHT = 128
WIDTH = 128
SIZE = 8
NEURONS = 16
NUM_W = WIDTH - (SIZE - 1)
NUM_H = HEIGHT - (SIZE - 1)
NUM_ZONES = NUM_H * NUM_W
PATCH = SIZE * SIZE
RB = NUM_W * NEURONS
NBUF = 11


def _tc_body(x_ref, w_hbm, o_ref, wbuf, sem):
    def cp(r, slot):
        return pltpu.make_async_copy(
            w_hbm.at[pl.ds(r * RB, RB)], wbuf.at[slot], sem.at[slot])

    for b in range(NBUF - 1):
        cp(b, b).start()

    def group_fn(g, carry):
        r0 = g * NBUF
        for b in range(NBUF):
            r = r0 + b
            nxt = r + NBUF - 1

            @pl.when(nxt < NUM_H)
            def _():
                cp(nxt, (b + NBUF - 1) % NBUF).start()

            cp(r, b).wait()
            wv = wbuf[b].reshape(NUM_W, NEURONS, PATCH)

            xs = x_ref[pl.ds(r, SIZE), :]
            segs = []
            for dr in range(SIZE):
                row = xs[dr:dr + 1, :]
                for dc in range(SIZE):
                    segs.append(row[:, dc:dc + NUM_W])
            PT = jnp.concatenate(segs, axis=0)
            P = PT.T
            prod = wv * P[:, None, :]
            resp = jnp.sum(prod, axis=2)
            m = jnp.max(resp, axis=1, keepdims=True)
            o_ref[r] = jnp.where(resp >= m, resp, 0.0)
        return carry

    jax.lax.fori_loop(0, NUM_H // NBUF, group_fn, 0)


def kernel(x, W):
    W2 = W.reshape(NUM_ZONES * NEURONS, PATCH)
    out = pl.pallas_call(
        _tc_body,
        in_specs=[
            pl.BlockSpec((HEIGHT, WIDTH), lambda: (0, 0)),
            pl.BlockSpec(memory_space=pl.ANY),
        ],
        out_specs=pl.BlockSpec((NUM_H, NUM_W, NEURONS), lambda: (0, 0, 0)),
        out_shape=jax.ShapeDtypeStruct((NUM_H, NUM_W, NEURONS), jnp.float32),
        scratch_shapes=[
            pltpu.VMEM((NBUF, RB, PATCH), jnp.float32),
            pltpu.SemaphoreType.DMA((NBUF,)),
        ],
    )(x, W2)
    return out.reshape(NUM_ZONES, NEURONS)
